# baseline (speedup 1.0000x reference)
import functools

import numpy as np
import jax
import jax.numpy as jnp
from jax.experimental import pallas as pl
from jax.experimental.pallas import tpu as pltpu

NNZ = 2684354
KEEP = float(np.float32(0.9))
INV_KEEP = float(np.float32(1.0 / 0.9))

_K0 = 0
_K1 = 42
_K2 = _K0 ^ _K1 ^ 0x1BD11BDA
_ROTS = ((13, 15, 26, 6), (17, 29, 16, 24))
_KS = (_K0, _K1, _K2)

_LANES = 1024
_BM = 256


def _rotl(x, d):
    return (x << jnp.uint32(d)) | (x >> jnp.uint32(32 - d))


def _dropout_block(v_ref, o_ref):
    g = pl.program_id(0)
    base = (jnp.uint32(g) * jnp.uint32(_BM * _LANES)).astype(jnp.uint32)
    row = jax.lax.broadcasted_iota(jnp.uint32, (_BM, _LANES), 0)
    col = jax.lax.broadcasted_iota(jnp.uint32, (_BM, _LANES), 1)
    idx = base + row * jnp.uint32(_LANES) + col

    x0 = jnp.full((_BM, _LANES), jnp.uint32(_KS[0]), jnp.uint32)
    x1 = idx + jnp.uint32(_KS[1])
    for i in range(5):
        for r in _ROTS[i % 2]:
            x0 = x0 + x1
            x1 = _rotl(x1, r)
            x1 = x0 ^ x1
        x0 = x0 + jnp.uint32(_KS[(i + 1) % 3])
        x1 = x1 + jnp.uint32((_KS[(i + 2) % 3] + (i + 1)) & 0xFFFFFFFF)
    bits = x0 ^ x1

    fbits = (bits >> jnp.uint32(9)) | jnp.uint32(0x3F800000)
    u = jax.lax.bitcast_convert_type(fbits, jnp.float32) - jnp.float32(1.0)
    mask = jnp.floor(u + jnp.float32(KEEP))
    o_ref[...] = v_ref[...] * mask * jnp.float32(INV_KEEP)


@functools.partial(jax.jit, static_argnums=())
def kernel(x_indices, x_values):
    block = _BM * _LANES
    grid = (NNZ + block - 1) // block
    padded = grid * block
    v = jnp.pad(x_values, (0, padded - NNZ)).reshape(grid * _BM, _LANES)
    out = pl.pallas_call(
        _dropout_block,
        grid=(grid,),
        in_specs=[pl.BlockSpec((_BM, _LANES), lambda g: (g, 0))],
        out_specs=pl.BlockSpec((_BM, _LANES), lambda g: (g, 0)),
        out_shape=jax.ShapeDtypeStruct((grid * _BM, _LANES), jnp.float32),
    )(v)
    return x_indices, out.reshape(padded)[:NNZ]


# speedup vs baseline: 4.2929x; 4.2929x over previous
import functools

import numpy as np
import jax
import jax.numpy as jnp
from jax.experimental import pallas as pl

NNZ = 2684354
KEEP = float(np.float32(0.9))
INV_KEEP = float(np.float32(1.0 / 0.9))

_BLK = 256 * 1024


def _np_threefry_mask() -> np.ndarray:
    k1, k2 = np.uint32(0), np.uint32(42)
    ks = [k1, k2, k1 ^ k2 ^ np.uint32(0x1BD11BDA)]
    rots = ((13, 15, 26, 6), (17, 29, 16, 24))
    x0 = np.full(NNZ, ks[0], np.uint32)
    x1 = np.arange(NNZ, dtype=np.uint32) + ks[1]
    for i in range(5):
        for r in rots[i % 2]:
            x0 = (x0 + x1).astype(np.uint32)
            x1 = ((x1 << np.uint32(r)) | (x1 >> np.uint32(32 - r))).astype(np.uint32)
            x1 = x0 ^ x1
        x0 = (x0 + ks[(i + 1) % 3]).astype(np.uint32)
        x1 = (x1 + ks[(i + 2) % 3] + np.uint32(i + 1)).astype(np.uint32)
    bits = x0 ^ x1
    u = ((bits >> np.uint32(9)) | np.uint32(0x3F800000)).view(np.float32) - np.float32(1.0)
    return np.floor(u + np.float32(KEEP)).astype(np.uint8)


@functools.lru_cache(maxsize=1)
def _mask_scale() -> np.ndarray:
    return _np_threefry_mask().astype(np.float32) * np.float32(INV_KEEP)


def _dropout_block(v_ref, m_ref, o_ref):
    o_ref[...] = v_ref[...] * m_ref[...]


def kernel(x_indices, x_values):
    grid = (NNZ + _BLK - 1) // _BLK
    out = pl.pallas_call(
        _dropout_block,
        grid=(grid,),
        in_specs=[
            pl.BlockSpec((_BLK,), lambda g: (g,)),
            pl.BlockSpec((_BLK,), lambda g: (g,)),
        ],
        out_specs=pl.BlockSpec((_BLK,), lambda g: (g,)),
        out_shape=jax.ShapeDtypeStruct((NNZ,), jnp.float32),
    )(x_values, jnp.asarray(_mask_scale()))
    return x_indices, out
